# Initial kernel scaffold; baseline (speedup 1.0000x reference)
import jax
import jax.numpy as jnp
from jax.experimental import pallas as pl


def kernel(input_ids, word_emb, pos_emb, type_emb, gamma, beta, bias_transform):
    raise NotImplementedError("write your pallas kernel here")


# speedup vs baseline: 1.7205x; 1.7205x over previous
import functools

import jax
import jax.numpy as jnp
from jax import lax
from jax.experimental import pallas as pl
from jax.experimental.pallas import tpu as pltpu
from jax.experimental.pallas import tpu_sc as plsc

VOCAB = 30522
HID = 768
MAXPOS = 512
B = 128
S = 512
NTOK = B * S
EPS = 1e-12

LANES = 16
NC = 2
NS = 16
NW = NC * NS
TPW = NTOK // NW
CHUNK = 64


def _sc_gather_sub(ids, wtab, btab):
    mesh = plsc.VectorSubcoreMesh(core_axis_name="c", subcore_axis_name="s")

    @functools.partial(
        pl.kernel,
        mesh=mesh,
        out_type=jax.ShapeDtypeStruct((NTOK, HID), jnp.float32),
        scratch_types=[
            pltpu.VMEM((CHUNK,), jnp.int32),
            pltpu.VMEM((CHUNK, HID), jnp.float32),
            pltpu.VMEM((CHUNK, HID), jnp.float32),
            pltpu.SemaphoreType.DMA,
            pltpu.SemaphoreType.DMA,
        ],
    )
    def k(ids_hbm, wtab_hbm, btab_hbm, out_hbm, idx_v, wbuf, bbuf, semw, semb):
        wid = lax.axis_index("s") * NC + lax.axis_index("c")
        base = wid * TPW

        def chunk_body(j, carry):
            row = base + j * CHUNK
            pltpu.sync_copy(ids_hbm.at[pl.ds(row, CHUNK)], idx_v)
            cw = pltpu.async_copy(wtab_hbm.at[idx_v], wbuf, semw)
            cb = pltpu.async_copy(btab_hbm.at[idx_v], bbuf, semb)
            cw.wait()
            cb.wait()

            def tok_body(t, c2):
                for kk in range(HID // LANES):
                    sl = pl.ds(kk * LANES, LANES)
                    wbuf[t, sl] = wbuf[t, sl] - bbuf[t, sl]
                return c2

            lax.fori_loop(0, CHUNK, tok_body, 0)
            pltpu.sync_copy(wbuf, out_hbm.at[pl.ds(row, CHUNK), :])
            return carry

        lax.fori_loop(0, TPW // CHUNK, chunk_body, 0)

    return k(ids, wtab, btab)


ROWS_PER_BLOCK = 2048


def _tc_ln_body(x_ref, pos_ref, type_ref, g_ref, b_ref, o_ref):
    x = x_ref[...].reshape(ROWS_PER_BLOCK // S, S, HID)
    e = x + pos_ref[...][None, :, :] + type_ref[...][None, :, :]
    mean = jnp.mean(e, axis=-1, keepdims=True)
    d = e - mean
    var = jnp.mean(d * d, axis=-1, keepdims=True)
    o = d * lax.rsqrt(var + EPS) * g_ref[...][None, :, :] + b_ref[...][None, :, :]
    o_ref[...] = o.reshape(ROWS_PER_BLOCK, HID)


def _tc_ln(x, pos_emb, type_row, gamma, beta):
    grid = (NTOK // ROWS_PER_BLOCK,)
    return pl.pallas_call(
        _tc_ln_body,
        grid=grid,
        in_specs=[
            pl.BlockSpec((ROWS_PER_BLOCK, HID), lambda i: (i, 0)),
            pl.BlockSpec((S, HID), lambda i: (0, 0)),
            pl.BlockSpec((1, HID), lambda i: (0, 0)),
            pl.BlockSpec((1, HID), lambda i: (0, 0)),
            pl.BlockSpec((1, HID), lambda i: (0, 0)),
        ],
        out_specs=pl.BlockSpec((ROWS_PER_BLOCK, HID), lambda i: (i, 0)),
        out_shape=jax.ShapeDtypeStruct((NTOK, HID), jnp.float32),
    )(x, pos_emb, type_row, gamma, beta)


@jax.jit
def kernel(input_ids, word_emb, pos_emb, type_emb, gamma, beta, bias_transform):
    ids = input_ids.reshape(-1).astype(jnp.int32)
    g = _sc_gather_sub(ids, word_emb, bias_transform)
    out = _tc_ln(
        g,
        pos_emb,
        type_emb[0:1, :],
        gamma.reshape(1, HID),
        beta.reshape(1, HID),
    )
    return out.reshape(B, S, HID)


# speedup vs baseline: 2.3515x; 1.3668x over previous
import functools

import jax
import jax.numpy as jnp
from jax import lax
from jax.experimental import pallas as pl
from jax.experimental.pallas import tpu as pltpu
from jax.experimental.pallas import tpu_sc as plsc

VOCAB = 30522
HID = 768
MAXPOS = 512
B = 128
S = 512
NTOK = B * S
EPS = 1e-12

LANES = 16
NC = 2
NS = 16
NW = NC * NS
TPW = NTOK // NW
CHUNK = 16
NBUF = 3
NCH = TPW // CHUNK


def _sc_gather_sub(ids, wtab, btab):
    mesh = plsc.VectorSubcoreMesh(core_axis_name="c", subcore_axis_name="s")

    scratch = [pltpu.VMEM((TPW,), jnp.int32)]
    scratch += [pltpu.VMEM((CHUNK, HID), jnp.float32) for _ in range(3 * NBUF)]
    scratch += [pltpu.SemaphoreType.DMA for _ in range(3 * NBUF)]

    @functools.partial(
        pl.kernel,
        mesh=mesh,
        out_type=jax.ShapeDtypeStruct((NTOK, HID), jnp.float32),
        scratch_types=scratch,
    )
    def k(ids_hbm, wtab_hbm, btab_hbm, out_hbm, idx_v, *rest):
        bufs = rest[: 3 * NBUF]
        sems = rest[3 * NBUF :]
        wbuf = bufs[0:NBUF]
        bbuf = bufs[NBUF : 2 * NBUF]
        obuf = bufs[2 * NBUF : 3 * NBUF]
        semw = sems[0:NBUF]
        semb = sems[NBUF : 2 * NBUF]
        semo = sems[2 * NBUF : 3 * NBUF]

        wid = lax.axis_index("s") * NC + lax.axis_index("c")
        base = wid * TPW
        pltpu.sync_copy(ids_hbm.at[pl.ds(base, TPW)], idx_v)

        def gathers(j, p):
            idx = idx_v.at[pl.ds(j * CHUNK, CHUNK)]
            pltpu.async_copy(wtab_hbm.at[idx], wbuf[p], semw[p])
            pltpu.async_copy(btab_hbm.at[idx], bbuf[p], semb[p])

        def wait_gathers(j, p):
            idx = idx_v.at[pl.ds(j * CHUNK, CHUNK)]
            pltpu.make_async_copy(wtab_hbm.at[idx], wbuf[p], semw[p]).wait()
            pltpu.make_async_copy(btab_hbm.at[idx], bbuf[p], semb[p]).wait()

        def out_region(j):
            return out_hbm.at[pl.ds(base + j * CHUNK, CHUNK), :]

        for p in range(NBUF):
            gathers(p, p)

        def group_body(jj, carry):
            for p in range(NBUF):
                j = jj * NBUF + p
                wait_gathers(j, p)

                @pl.when(jj > 0)
                def _():
                    pltpu.make_async_copy(obuf[p], out_region(j), semo[p]).wait()

                def tok_body(t, c2):
                    for kk in range(HID // LANES):
                        sl = pl.ds(kk * LANES, LANES)
                        obuf[p][t, sl] = wbuf[p][t, sl] - bbuf[p][t, sl]
                    return c2

                lax.fori_loop(0, CHUNK, tok_body, 0)
                pltpu.async_copy(obuf[p], out_region(j), semo[p])

                @pl.when(j + NBUF < NCH)
                def _():
                    gathers(j + NBUF, p)
            return carry

        lax.fori_loop(0, NCH // NBUF, group_body, 0)
        for p in range(NCH % NBUF):
            j = (NCH // NBUF) * NBUF + p
            wait_gathers(j, p)
            pltpu.make_async_copy(obuf[p], out_region(j), semo[p]).wait()

            def tok_body(t, c2):
                for kk in range(HID // LANES):
                    sl = pl.ds(kk * LANES, LANES)
                    obuf[p][t, sl] = wbuf[p][t, sl] - bbuf[p][t, sl]
                return c2

            lax.fori_loop(0, CHUNK, tok_body, 0)
            pltpu.async_copy(obuf[p], out_region(j), semo[p])

        for p in range(NBUF):
            pltpu.make_async_copy(obuf[p], out_region(p), semo[p]).wait()

    return k(ids, wtab, btab)


ROWS_PER_BLOCK = 2048


def _tc_ln_body(x_ref, pos_ref, type_ref, g_ref, b_ref, o_ref):
    x = x_ref[...].reshape(ROWS_PER_BLOCK // S, S, HID)
    e = x + pos_ref[...][None, :, :] + type_ref[...][None, :, :]
    mean = jnp.mean(e, axis=-1, keepdims=True)
    d = e - mean
    var = jnp.mean(d * d, axis=-1, keepdims=True)
    o = d * lax.rsqrt(var + EPS) * g_ref[...][None, :, :] + b_ref[...][None, :, :]
    o_ref[...] = o.reshape(ROWS_PER_BLOCK, HID)


def _tc_ln(x, pos_emb, type_row, gamma, beta):
    grid = (NTOK // ROWS_PER_BLOCK,)
    return pl.pallas_call(
        _tc_ln_body,
        grid=grid,
        in_specs=[
            pl.BlockSpec((ROWS_PER_BLOCK, HID), lambda i: (i, 0)),
            pl.BlockSpec((S, HID), lambda i: (0, 0)),
            pl.BlockSpec((1, HID), lambda i: (0, 0)),
            pl.BlockSpec((1, HID), lambda i: (0, 0)),
            pl.BlockSpec((1, HID), lambda i: (0, 0)),
        ],
        out_specs=pl.BlockSpec((ROWS_PER_BLOCK, HID), lambda i: (i, 0)),
        out_shape=jax.ShapeDtypeStruct((NTOK, HID), jnp.float32),
    )(x, pos_emb, type_row, gamma, beta)


@jax.jit
def kernel(input_ids, word_emb, pos_emb, type_emb, gamma, beta, bias_transform):
    ids = input_ids.reshape(-1).astype(jnp.int32)
    g = _sc_gather_sub(ids, word_emb, bias_transform)
    out = _tc_ln(
        g,
        pos_emb,
        type_emb[0:1, :],
        gamma.reshape(1, HID),
        beta.reshape(1, HID),
    )
    return out.reshape(B, S, HID)
